# baseline (speedup 1.0000x reference)
import functools

import jax
import jax.numpy as jnp
from jax import lax
from jax.experimental import pallas as pl
from jax.experimental.pallas import tpu as pltpu
from jax.experimental.pallas import tpu_sc as plsc

N_SRC = 10000
N_DST = 10000
CH = 128
E = 320000
B = 128
NC = 2
NS = 16
NW = NC * NS
NBLK = 10
BCH = 8
KTOT = NBLK * BCH
EPAD = KTOT * NW * B
ROWS_PAD = 10240
ZCH = B
NZ = ROWS_PAD // NS // ZCH

_mesh = plsc.VectorSubcoreMesh(
    core_axis_name="c", subcore_axis_name="s", num_cores=NC, num_subcores=NS)


@functools.partial(
    pl.kernel,
    out_type=(
        jax.ShapeDtypeStruct((NC, ROWS_PAD, CH), jnp.float32),
        jax.ShapeDtypeStruct((NW, ROWS_PAD), jnp.float32),
    ),
    mesh=_mesh,
    scratch_types=[
        pltpu.VMEM_SHARED((ROWS_PAD, CH), jnp.float32),
        pltpu.VMEM((2, BCH, B), jnp.int32),
        pltpu.VMEM((2, BCH, B), jnp.int32),
        pltpu.VMEM((2, B, CH), jnp.float32),
        pltpu.VMEM((ROWS_PAD,), jnp.float32),
        [pltpu.SemaphoreType.DMA] * 2,
        [pltpu.SemaphoreType.DMA] * 2,
    ],
    compiler_params=pltpu.CompilerParams(needs_layout_passes=False),
)
def _sc_aggregate(x_hbm, src_hbm, dst_hbm, out_hbm, cnt_hbm,
                  acc, idx_s, idx_d, rows, cnt, semi, semg):
    c = lax.axis_index("c")
    s = lax.axis_index("s")
    wid = s * NC + c

    zeros16 = jnp.zeros((16,), jnp.float32)

    def zrow(r, carry):
        for j in range(CH // 16):
            rows[0, r, pl.ds(j * 16, 16)] = zeros16
        return carry
    lax.fori_loop(0, B, zrow, 0)

    def zcnt(r, carry):
        for j in range(4):
            cnt[pl.ds(r * 64 + j * 16, 16)] = zeros16
        return carry
    lax.fori_loop(0, ROWS_PAD // 64, zcnt, 0)

    for j in range(NZ):
        r0 = s * (ROWS_PAD // NS) + j * ZCH
        pltpu.sync_copy(rows.at[0], acc.at[pl.ds(r0, ZCH), :])
    plsc.subcore_barrier()

    ones16 = jnp.ones((16,), jnp.float32)

    def issue_idx(q, blk):
        pltpu.async_copy(src_hbm.at[pl.ds(blk * BCH, BCH), wid, :],
                         idx_s.at[q], semi[q])
        pltpu.async_copy(dst_hbm.at[pl.ds(blk * BCH, BCH), wid, :],
                         idx_d.at[q], semi[q])

    def wait_idx(q):
        pltpu.make_async_copy(src_hbm.at[pl.ds(0, BCH), wid, :],
                              idx_s.at[q], semi[q]).wait()
        pltpu.make_async_copy(dst_hbm.at[pl.ds(0, BCH), wid, :],
                              idx_d.at[q], semi[q]).wait()

    def issue_gather(q, j, b):
        pltpu.async_copy(x_hbm.at[idx_s.at[q, j]], rows.at[b], semg[b])

    def wait_gather(q, j, b):
        pltpu.make_async_copy(x_hbm.at[idx_s.at[q, j]], rows.at[b],
                              semg[b]).wait()

    def section(t, q, j):
        b = j % 2

        if j == 6:
            if q == 0:
                wait_idx(1)
            else:
                @pl.when(t < NBLK // 2 - 1)
                def _():
                    wait_idx(0)

        if j < BCH - 1:
            issue_gather(q, j + 1, 1 - b)
        elif q == 0:
            issue_gather(1, 0, 1 - b)
        else:
            @pl.when(t < NBLK // 2 - 1)
            def _():
                issue_gather(0, 0, 1 - b)

        for i in range(B // 16):
            d16 = idx_d[q, j, pl.ds(i * 16, 16)]
            plsc.addupdate_scatter(cnt, [d16], ones16)

        wait_gather(q, j, b)
        pltpu.sync_copy(rows.at[b], acc.at[idx_d.at[q, j]], add=True)

        if j == BCH - 1:
            @pl.when(t < NBLK // 2 - 1)
            def _():
                issue_idx(q, 2 * t + q + 2)

    issue_idx(0, 0)
    wait_idx(0)
    issue_idx(1, 1)
    issue_gather(0, 0, 0)

    def outer(t, carry):
        for q in range(2):
            for j in range(BCH):
                section(t, q, j)
        return carry
    lax.fori_loop(0, NBLK // 2, outer, 0)

    pltpu.sync_copy(cnt, cnt_hbm.at[wid])

    plsc.subcore_barrier()

    for j in range(NZ):
        r0 = s * (ROWS_PAD // NS) + j * ZCH
        pltpu.sync_copy(acc.at[pl.ds(r0, ZCH), :], rows.at[0])
        pltpu.sync_copy(rows.at[0], out_hbm.at[c, pl.ds(r0, ZCH), :])


def _finish_body(a_ref, c_ref, w_ref, b_ref, o_ref):
    a = a_ref[0] + a_ref[1]
    cnt = jnp.sum(c_ref[...], axis=0)[:, None]
    y = jnp.dot(a, w_ref[...], preferred_element_type=jnp.float32)
    y = y * lax.rsqrt(1.0 + cnt) + b_ref[...]
    o_ref[...] = jnp.maximum(y, 0.0)


RBLK = 512

_finish = pl.pallas_call(
    _finish_body,
    grid=(ROWS_PAD // RBLK,),
    in_specs=[
        pl.BlockSpec((NC, RBLK, CH), lambda i: (0, i, 0)),
        pl.BlockSpec((NW, RBLK), lambda i: (0, i)),
        pl.BlockSpec((CH, CH), lambda i: (0, 0)),
        pl.BlockSpec((1, CH), lambda i: (0, 0)),
    ],
    out_specs=pl.BlockSpec((RBLK, CH), lambda i: (i, 0)),
    out_shape=jax.ShapeDtypeStruct((ROWS_PAD, CH), jnp.float32),
)


def kernel(x, edge_index, W, b):
    x = x.astype(jnp.float32)
    pad = EPAD - E
    src = jnp.concatenate(
        [edge_index[0].astype(jnp.int32),
         jnp.zeros((pad,), jnp.int32)]).reshape(KTOT, NW, B)
    dst = jnp.concatenate(
        [edge_index[1].astype(jnp.int32),
         jnp.full((pad,), N_DST, jnp.int32)]).reshape(KTOT, NW, B)
    partials, counts = _sc_aggregate(x, src, dst)
    out = _finish(partials, counts, W.astype(jnp.float32),
                  b.astype(jnp.float32).reshape(1, CH))
    return out[:N_DST]


# speedup vs baseline: 2.8575x; 2.8575x over previous
import functools

import jax
import jax.numpy as jnp
from jax import lax
from jax.experimental import pallas as pl
from jax.experimental.pallas import tpu as pltpu
from jax.experimental.pallas import tpu_sc as plsc

N_SRC = 10000
N_DST = 10000
CH = 128
E = 320000
B = 128
NCHUNK = E // B
NC = 2
NS = 16
NW = NC * NS
ROWS_PAD = 10240
ZCH = ROWS_PAD // NS // 5
KMAX = (NCHUNK + NW - 1) // NW

_mesh = plsc.VectorSubcoreMesh(
    core_axis_name="c", subcore_axis_name="s", num_cores=NC, num_subcores=NS)


@functools.partial(
    pl.kernel,
    out_type=(
        jax.ShapeDtypeStruct((NC, ROWS_PAD, CH), jnp.float32),
        jax.ShapeDtypeStruct((NW, ROWS_PAD), jnp.float32),
    ),
    mesh=_mesh,
    scratch_types=[
        pltpu.VMEM_SHARED((ROWS_PAD, CH), jnp.float32),
        pltpu.VMEM((2, B), jnp.int32),
        pltpu.VMEM((2, B), jnp.int32),
        pltpu.VMEM((2, B, CH), jnp.float32),
        pltpu.VMEM((ROWS_PAD,), jnp.float32),
        pltpu.SemaphoreType.DMA,
        pltpu.SemaphoreType.DMA,
        pltpu.SemaphoreType.DMA,
        pltpu.SemaphoreType.DMA,
    ],
    compiler_params=pltpu.CompilerParams(needs_layout_passes=False),
)
def _sc_aggregate(x_hbm, edge_hbm, out_hbm, cnt_hbm,
                  acc, idx_s, idx_d, rows, cnt,
                  semi0, semi1, semg0, semg1):
    c = lax.axis_index("c")
    s = lax.axis_index("s")
    wid = s * NC + c
    semi = (semi0, semi1)
    semg = (semg0, semg1)

    zeros16 = jnp.zeros((16,), jnp.float32)

    def zrow(r, carry):
        for j in range(CH // 16):
            rows[0, r, pl.ds(j * 16, 16)] = zeros16
        return carry
    lax.fori_loop(0, B, zrow, 0)

    def zcnt(r, carry):
        for j in range(4):
            cnt[pl.ds(r * 64 + j * 16, 16)] = zeros16
        return carry
    lax.fori_loop(0, ROWS_PAD // 64, zcnt, 0)

    for j in range(ROWS_PAD // NS // ZCH):
        r0 = s * (ROWS_PAD // NS) + j * ZCH
        pltpu.async_copy(rows.at[0], acc.at[pl.ds(r0, ZCH), :], semg0)
    for j in range(ROWS_PAD // NS // ZCH):
        r0 = s * (ROWS_PAD // NS) + j * ZCH
        pltpu.make_async_copy(rows.at[0], acc.at[pl.ds(r0, ZCH), :],
                              semg0).wait()
    plsc.subcore_barrier()

    ones16 = jnp.ones((16,), jnp.float32)

    def issue_idx(b, kk):
        base = (wid + kk * NW) * B
        pltpu.async_copy(edge_hbm.at[0, pl.ds(base, B)], idx_s.at[b], semi[b])
        pltpu.async_copy(edge_hbm.at[1, pl.ds(base, B)], idx_d.at[b], semi[b])

    def wait_idx(b):
        pltpu.make_async_copy(edge_hbm.at[0, pl.ds(0, B)], idx_s.at[b],
                              semi[b]).wait()
        pltpu.make_async_copy(edge_hbm.at[1, pl.ds(0, B)], idx_d.at[b],
                              semi[b]).wait()

    def issue_gather(b):
        pltpu.async_copy(x_hbm.at[idx_s.at[b]], rows.at[b], semg[b])

    def wait_gather(b):
        pltpu.make_async_copy(x_hbm.at[idx_s.at[b]], rows.at[b],
                              semg[b]).wait()

    def section(b, kk, chunk):
        @pl.when(chunk < NCHUNK)
        def _():
            nb = 1 - b

            @pl.when(chunk + NW < NCHUNK)
            def _():
                wait_idx(nb)
                issue_gather(nb)
            for j in range(B // 16):
                d16 = idx_d[b, pl.ds(j * 16, 16)]
                plsc.addupdate_scatter(cnt, [d16], ones16)
            wait_gather(b)
            pltpu.sync_copy(rows.at[b], acc.at[idx_d.at[b]], add=True)

            @pl.when(chunk + 2 * NW < NCHUNK)
            def _():
                issue_idx(b, kk + 2)

    issue_idx(0, 0)
    wait_idx(0)
    issue_gather(0)

    @pl.when(wid + NW < NCHUNK)
    def _():
        issue_idx(1, 1)

    def outer(t, carry):
        kk0 = 2 * t
        section(0, kk0, wid + kk0 * NW)
        section(1, kk0 + 1, wid + (kk0 + 1) * NW)
        return carry
    lax.fori_loop(0, (KMAX + 1) // 2, outer, 0)

    cnt_out = pltpu.async_copy(cnt, cnt_hbm.at[wid], semi0)

    plsc.subcore_barrier()

    for j in range(ROWS_PAD // NS // ZCH):
        b = j % 2
        r0 = s * (ROWS_PAD // NS) + j * ZCH
        if j >= 2:
            r0p = s * (ROWS_PAD // NS) + (j - 2) * ZCH
            pltpu.make_async_copy(rows.at[b],
                                  out_hbm.at[c, pl.ds(r0p, ZCH), :],
                                  semg[b]).wait()
        pltpu.sync_copy(acc.at[pl.ds(r0, ZCH), :], rows.at[b])
        pltpu.async_copy(rows.at[b], out_hbm.at[c, pl.ds(r0, ZCH), :],
                         semg[b])
    for j in (3, 4):
        b = j % 2
        r0 = s * (ROWS_PAD // NS) + j * ZCH
        pltpu.make_async_copy(rows.at[b], out_hbm.at[c, pl.ds(r0, ZCH), :],
                              semg[b]).wait()
    cnt_out.wait()


def _finish_body(a_ref, c_ref, w_ref, b_ref, o_ref):
    a = a_ref[0] + a_ref[1]
    cnt = jnp.sum(c_ref[...], axis=0)[:, None]
    y = jnp.dot(a, w_ref[...], preferred_element_type=jnp.float32)
    y = y * lax.rsqrt(1.0 + cnt) + b_ref[...]
    o_ref[...] = jnp.maximum(y, 0.0)


RBLK = 512

_finish = pl.pallas_call(
    _finish_body,
    grid=(ROWS_PAD // RBLK,),
    in_specs=[
        pl.BlockSpec((NC, RBLK, CH), lambda i: (0, i, 0)),
        pl.BlockSpec((NW, RBLK), lambda i: (0, i)),
        pl.BlockSpec((CH, CH), lambda i: (0, 0)),
        pl.BlockSpec((1, CH), lambda i: (0, 0)),
    ],
    out_specs=pl.BlockSpec((RBLK, CH), lambda i: (i, 0)),
    out_shape=jax.ShapeDtypeStruct((ROWS_PAD, CH), jnp.float32),
)


def kernel(x, edge_index, W, b):
    x = x.astype(jnp.float32)
    partials, counts = _sc_aggregate(x, edge_index.astype(jnp.int32))
    out = _finish(partials, counts,
                  W.astype(jnp.float32), b.astype(jnp.float32).reshape(1, CH))
    return out[:N_DST]


# speedup vs baseline: 2.9477x; 1.0315x over previous
import functools

import jax
import jax.numpy as jnp
from jax import lax
from jax.experimental import pallas as pl
from jax.experimental.pallas import tpu as pltpu
from jax.experimental.pallas import tpu_sc as plsc

N_SRC = 10000
N_DST = 10000
CH = 128
E = 320000
B = 128
NCHUNK = E // B
NC = 2
NS = 16
NW = NC * NS
ROWS_PAD = 10240
ZCH = ROWS_PAD // NS // 5
KMAX = (NCHUNK + NW - 1) // NW

_mesh = plsc.VectorSubcoreMesh(
    core_axis_name="c", subcore_axis_name="s", num_cores=NC, num_subcores=NS)


@functools.partial(
    pl.kernel,
    out_type=(
        jax.ShapeDtypeStruct((NC, ROWS_PAD, CH), jnp.float32),
        jax.ShapeDtypeStruct((NW, ROWS_PAD), jnp.float32),
    ),
    mesh=_mesh,
    scratch_types=[
        pltpu.VMEM_SHARED((ROWS_PAD, CH), jnp.float32),
        pltpu.VMEM((2, B), jnp.int32),
        pltpu.VMEM((2, B), jnp.int32),
        pltpu.VMEM((2, B, CH), jnp.float32),
        pltpu.VMEM((ROWS_PAD,), jnp.float32),
        pltpu.SemaphoreType.DMA,
        pltpu.SemaphoreType.DMA,
        pltpu.SemaphoreType.DMA,
        pltpu.SemaphoreType.DMA,
    ],
    compiler_params=pltpu.CompilerParams(needs_layout_passes=False),
)
def _sc_aggregate(x_hbm, edge_hbm, out_hbm, cnt_hbm,
                  acc, idx_s, idx_d, rows, cnt,
                  semi0, semi1, semg0, semg1):
    c = lax.axis_index("c")
    s = lax.axis_index("s")
    wid = s * NC + c
    semi = (semi0, semi1)
    semg = (semg0, semg1)

    zeros16 = jnp.zeros((16,), jnp.float32)

    def zrow(r, carry):
        for j in range(CH // 16):
            rows[0, r, pl.ds(j * 16, 16)] = zeros16
        return carry
    lax.fori_loop(0, B, zrow, 0)

    def zcnt(r, carry):
        for j in range(4):
            cnt[pl.ds(r * 64 + j * 16, 16)] = zeros16
        return carry
    lax.fori_loop(0, ROWS_PAD // 64, zcnt, 0)

    for j in range(ROWS_PAD // NS // ZCH):
        r0 = s * (ROWS_PAD // NS) + j * ZCH
        pltpu.async_copy(rows.at[0], acc.at[pl.ds(r0, ZCH), :], semg0)
    for j in range(ROWS_PAD // NS // ZCH):
        r0 = s * (ROWS_PAD // NS) + j * ZCH
        pltpu.make_async_copy(rows.at[0], acc.at[pl.ds(r0, ZCH), :],
                              semg0).wait()
    plsc.subcore_barrier()

    ones16 = jnp.ones((16,), jnp.float32)

    def issue_idx(b, kk):
        base = (wid + kk * NW) * B
        pltpu.async_copy(edge_hbm.at[0, pl.ds(base, B)], idx_s.at[b], semi[b])
        pltpu.async_copy(edge_hbm.at[1, pl.ds(base, B)], idx_d.at[b], semi[b])

    def wait_idx(b):
        pltpu.make_async_copy(edge_hbm.at[0, pl.ds(0, B)], idx_s.at[b],
                              semi[b]).wait()
        pltpu.make_async_copy(edge_hbm.at[1, pl.ds(0, B)], idx_d.at[b],
                              semi[b]).wait()

    def issue_gather(b):
        pltpu.async_copy(x_hbm.at[idx_s.at[b]], rows.at[b], semg[b])

    def wait_gather(b):
        pltpu.make_async_copy(x_hbm.at[idx_s.at[b]], rows.at[b],
                              semg[b]).wait()

    def section(b, kk, chunk):
        @pl.when(chunk < NCHUNK)
        def _():
            nb = 1 - b

            @pl.when(chunk + NW < NCHUNK)
            def _():
                wait_idx(nb)
                issue_gather(nb)
            for j in range(B // 16):
                d16 = idx_d[b, pl.ds(j * 16, 16)]
                plsc.addupdate_scatter(cnt, [d16], ones16)
            wait_gather(b)
            pltpu.sync_copy(rows.at[b], acc.at[idx_d.at[b]], add=True)

            @pl.when(chunk + 2 * NW < NCHUNK)
            def _():
                issue_idx(b, kk + 2)

    issue_idx(0, 0)
    wait_idx(0)
    issue_gather(0)

    @pl.when(wid + NW < NCHUNK)
    def _():
        issue_idx(1, 1)

    def outer(t, carry):
        kk0 = 2 * t
        section(0, kk0, wid + kk0 * NW)
        section(1, kk0 + 1, wid + (kk0 + 1) * NW)
        return carry
    lax.fori_loop(0, (KMAX + 1) // 2, outer, 0)

    cnt_out = pltpu.async_copy(cnt, cnt_hbm.at[wid], semi0)

    plsc.subcore_barrier()

    for j in range(ROWS_PAD // NS // ZCH):
        b = j % 2
        r0 = s * (ROWS_PAD // NS) + j * ZCH
        if j >= 2:
            r0p = s * (ROWS_PAD // NS) + (j - 2) * ZCH
            pltpu.make_async_copy(rows.at[b],
                                  out_hbm.at[c, pl.ds(r0p, ZCH), :],
                                  semg[b]).wait()
        pltpu.sync_copy(acc.at[pl.ds(r0, ZCH), :], rows.at[b])
        pltpu.async_copy(rows.at[b], out_hbm.at[c, pl.ds(r0, ZCH), :],
                         semg[b])
    for j in (3, 4):
        b = j % 2
        r0 = s * (ROWS_PAD // NS) + j * ZCH
        pltpu.make_async_copy(rows.at[b], out_hbm.at[c, pl.ds(r0, ZCH), :],
                              semg[b]).wait()
    cnt_out.wait()


def _finish_body(a_ref, c_ref, w_ref, b_ref, o_ref):
    a = a_ref[0] + a_ref[1]
    cnt = jnp.sum(c_ref[...], axis=0)[:, None]
    y = jnp.dot(a, w_ref[...], preferred_element_type=jnp.float32)
    y = y * lax.rsqrt(1.0 + cnt) + b_ref[...]
    o_ref[...] = jnp.maximum(y, 0.0)


RBLK = 1024

_finish = pl.pallas_call(
    _finish_body,
    grid=(ROWS_PAD // RBLK,),
    in_specs=[
        pl.BlockSpec((NC, RBLK, CH), lambda i: (0, i, 0)),
        pl.BlockSpec((NW, RBLK), lambda i: (0, i)),
        pl.BlockSpec((CH, CH), lambda i: (0, 0)),
        pl.BlockSpec((1, CH), lambda i: (0, 0)),
    ],
    out_specs=pl.BlockSpec((RBLK, CH), lambda i: (i, 0)),
    out_shape=jax.ShapeDtypeStruct((ROWS_PAD, CH), jnp.float32),
)


def kernel(x, edge_index, W, b):
    x = x.astype(jnp.float32)
    partials, counts = _sc_aggregate(x, edge_index.astype(jnp.int32))
    out = _finish(partials, counts,
                  W.astype(jnp.float32), b.astype(jnp.float32).reshape(1, CH))
    return out[:N_DST]


# speedup vs baseline: 2.9956x; 1.0163x over previous
import functools

import jax
import jax.numpy as jnp
from jax import lax
from jax.experimental import pallas as pl
from jax.experimental.pallas import tpu as pltpu
from jax.experimental.pallas import tpu_sc as plsc

N_SRC = 10000
N_DST = 10000
CH = 128
E = 320000
B = 128
NCHUNK = E // B
NC = 2
NS = 16
NW = NC * NS
ROWS_PAD = 10240
ZCH = ROWS_PAD // NS // 5
KMAX = (NCHUNK + NW - 1) // NW

_mesh = plsc.VectorSubcoreMesh(
    core_axis_name="c", subcore_axis_name="s", num_cores=NC, num_subcores=NS)


@functools.partial(
    pl.kernel,
    out_type=(
        jax.ShapeDtypeStruct((NC, ROWS_PAD, CH), jnp.float32),
        jax.ShapeDtypeStruct((NW, ROWS_PAD), jnp.float32),
    ),
    mesh=_mesh,
    scratch_types=[
        pltpu.VMEM_SHARED((ROWS_PAD, CH), jnp.float32),
        pltpu.VMEM((2, B), jnp.int32),
        pltpu.VMEM((2, B), jnp.int32),
        pltpu.VMEM((2, B, CH), jnp.float32),
        pltpu.VMEM((ROWS_PAD,), jnp.float32),
        pltpu.SemaphoreType.DMA,
        pltpu.SemaphoreType.DMA,
        pltpu.SemaphoreType.DMA,
        pltpu.SemaphoreType.DMA,
    ],
    compiler_params=pltpu.CompilerParams(needs_layout_passes=False),
)
def _sc_aggregate(x_hbm, edge_hbm, out_hbm, cnt_hbm,
                  acc, idx_s, idx_d, rows, cnt,
                  semi0, semi1, semg0, semg1):
    c = lax.axis_index("c")
    s = lax.axis_index("s")
    wid = s * NC + c
    semi = (semi0, semi1)
    semg = (semg0, semg1)

    zeros16 = jnp.zeros((16,), jnp.float32)

    def zrow(r, carry):
        for j in range(CH // 16):
            rows[0, r, pl.ds(j * 16, 16)] = zeros16
        return carry
    lax.fori_loop(0, B, zrow, 0)

    def zcnt(r, carry):
        for j in range(4):
            cnt[pl.ds(r * 64 + j * 16, 16)] = zeros16
        return carry
    lax.fori_loop(0, ROWS_PAD // 64, zcnt, 0)

    for j in range(ROWS_PAD // NS // ZCH):
        r0 = s * (ROWS_PAD // NS) + j * ZCH
        pltpu.async_copy(rows.at[0], acc.at[pl.ds(r0, ZCH), :], semg0)
    for j in range(ROWS_PAD // NS // ZCH):
        r0 = s * (ROWS_PAD // NS) + j * ZCH
        pltpu.make_async_copy(rows.at[0], acc.at[pl.ds(r0, ZCH), :],
                              semg0).wait()
    plsc.subcore_barrier()

    ones16 = jnp.ones((16,), jnp.float32)

    def issue_idx(b, kk):
        base = (wid + kk * NW) * B
        pltpu.async_copy(edge_hbm.at[0, pl.ds(base, B)], idx_s.at[b], semi[b])
        pltpu.async_copy(edge_hbm.at[1, pl.ds(base, B)], idx_d.at[b], semi[b])

    def wait_idx(b):
        pltpu.make_async_copy(edge_hbm.at[0, pl.ds(0, B)], idx_s.at[b],
                              semi[b]).wait()
        pltpu.make_async_copy(edge_hbm.at[1, pl.ds(0, B)], idx_d.at[b],
                              semi[b]).wait()

    def issue_gather(b):
        pltpu.async_copy(x_hbm.at[idx_s.at[b]], rows.at[b], semg[b])

    def wait_gather(b):
        pltpu.make_async_copy(x_hbm.at[idx_s.at[b]], rows.at[b],
                              semg[b]).wait()

    def section(b, kk, chunk):
        @pl.when(chunk < NCHUNK)
        def _():
            nb = 1 - b

            @pl.when(chunk + NW < NCHUNK)
            def _():
                wait_idx(nb)
                issue_gather(nb)
            for j in range(B // 16):
                d16 = idx_d[b, pl.ds(j * 16, 16)]
                plsc.addupdate_scatter(cnt, [d16], ones16)
            wait_gather(b)
            pltpu.sync_copy(rows.at[b], acc.at[idx_d.at[b]], add=True)

            @pl.when(chunk + 2 * NW < NCHUNK)
            def _():
                issue_idx(b, kk + 2)

    issue_idx(0, 0)
    wait_idx(0)
    issue_gather(0)

    @pl.when(wid + NW < NCHUNK)
    def _():
        issue_idx(1, 1)

    def outer(t, carry):
        kk0 = 2 * t
        section(0, kk0, wid + kk0 * NW)
        section(1, kk0 + 1, wid + (kk0 + 1) * NW)
        return carry
    lax.fori_loop(0, (KMAX + 1) // 2, outer, 0)

    cnt_out = pltpu.async_copy(cnt, cnt_hbm.at[wid], semi0)

    plsc.subcore_barrier()

    for j in range(ROWS_PAD // NS // ZCH):
        b = j % 2
        r0 = s * (ROWS_PAD // NS) + j * ZCH
        if j >= 2:
            r0p = s * (ROWS_PAD // NS) + (j - 2) * ZCH
            pltpu.make_async_copy(rows.at[b],
                                  out_hbm.at[c, pl.ds(r0p, ZCH), :],
                                  semg[b]).wait()
        pltpu.sync_copy(acc.at[pl.ds(r0, ZCH), :], rows.at[b])
        pltpu.async_copy(rows.at[b], out_hbm.at[c, pl.ds(r0, ZCH), :],
                         semg[b])
    for j in (3, 4):
        b = j % 2
        r0 = s * (ROWS_PAD // NS) + j * ZCH
        pltpu.make_async_copy(rows.at[b], out_hbm.at[c, pl.ds(r0, ZCH), :],
                              semg[b]).wait()
    cnt_out.wait()


def _finish_body(a_ref, c_ref, w_ref, b_ref, o_ref):
    a = a_ref[0] + a_ref[1]
    cnt = jnp.sum(c_ref[...], axis=0)[:, None]
    y = jnp.dot(a, w_ref[...], preferred_element_type=jnp.float32)
    y = y * lax.rsqrt(1.0 + cnt) + b_ref[...]
    o_ref[...] = jnp.maximum(y, 0.0)


RBLK = 2048

_finish = pl.pallas_call(
    _finish_body,
    grid=(ROWS_PAD // RBLK,),
    in_specs=[
        pl.BlockSpec((NC, RBLK, CH), lambda i: (0, i, 0)),
        pl.BlockSpec((NW, RBLK), lambda i: (0, i)),
        pl.BlockSpec((CH, CH), lambda i: (0, 0)),
        pl.BlockSpec((1, CH), lambda i: (0, 0)),
    ],
    out_specs=pl.BlockSpec((RBLK, CH), lambda i: (i, 0)),
    out_shape=jax.ShapeDtypeStruct((ROWS_PAD, CH), jnp.float32),
)


def kernel(x, edge_index, W, b):
    x = x.astype(jnp.float32)
    partials, counts = _sc_aggregate(x, edge_index.astype(jnp.int32))
    out = _finish(partials, counts,
                  W.astype(jnp.float32), b.astype(jnp.float32).reshape(1, CH))
    return out[:N_DST]


# speedup vs baseline: 3.0380x; 1.0142x over previous
import functools

import jax
import jax.numpy as jnp
from jax import lax
from jax.experimental import pallas as pl
from jax.experimental.pallas import tpu as pltpu
from jax.experimental.pallas import tpu_sc as plsc

N_SRC = 10000
N_DST = 10000
CH = 128
E = 320000
B = 128
NCHUNK = E // B
NC = 2
NS = 16
NW = NC * NS
ROWS_PAD = 10240
ZCH = ROWS_PAD // NS // 5
KMAX = (NCHUNK + NW - 1) // NW

_mesh = plsc.VectorSubcoreMesh(
    core_axis_name="c", subcore_axis_name="s", num_cores=NC, num_subcores=NS)


@functools.partial(
    pl.kernel,
    out_type=(
        jax.ShapeDtypeStruct((NC, ROWS_PAD, CH), jnp.float32),
        jax.ShapeDtypeStruct((NW, ROWS_PAD), jnp.float32),
    ),
    mesh=_mesh,
    scratch_types=[
        pltpu.VMEM_SHARED((ROWS_PAD, CH), jnp.float32),
        pltpu.VMEM((2, B), jnp.int32),
        pltpu.VMEM((2, B), jnp.int32),
        pltpu.VMEM((2, B, CH), jnp.float32),
        pltpu.VMEM((ROWS_PAD,), jnp.float32),
        pltpu.SemaphoreType.DMA,
        pltpu.SemaphoreType.DMA,
        pltpu.SemaphoreType.DMA,
        pltpu.SemaphoreType.DMA,
    ],
    compiler_params=pltpu.CompilerParams(needs_layout_passes=False),
)
def _sc_aggregate(x_hbm, edge_hbm, out_hbm, cnt_hbm,
                  acc, idx_s, idx_d, rows, cnt,
                  semi0, semi1, semg0, semg1):
    c = lax.axis_index("c")
    s = lax.axis_index("s")
    wid = s * NC + c
    semi = (semi0, semi1)
    semg = (semg0, semg1)

    zeros16 = jnp.zeros((16,), jnp.float32)

    def zrow(r, carry):
        for j in range(CH // 16):
            rows[0, r, pl.ds(j * 16, 16)] = zeros16
        return carry
    lax.fori_loop(0, B, zrow, 0)

    def zcnt(r, carry):
        for j in range(4):
            cnt[pl.ds(r * 64 + j * 16, 16)] = zeros16
        return carry
    lax.fori_loop(0, ROWS_PAD // 64, zcnt, 0)

    for j in range(ROWS_PAD // NS // ZCH):
        r0 = s * (ROWS_PAD // NS) + j * ZCH
        pltpu.async_copy(rows.at[0], acc.at[pl.ds(r0, ZCH), :], semg0)
    for j in range(ROWS_PAD // NS // ZCH):
        r0 = s * (ROWS_PAD // NS) + j * ZCH
        pltpu.make_async_copy(rows.at[0], acc.at[pl.ds(r0, ZCH), :],
                              semg0).wait()
    plsc.subcore_barrier()

    ones16 = jnp.ones((16,), jnp.float32)

    def issue_idx(b, kk):
        base = (wid + kk * NW) * B
        pltpu.async_copy(edge_hbm.at[0, pl.ds(base, B)], idx_s.at[b], semi[b])
        pltpu.async_copy(edge_hbm.at[1, pl.ds(base, B)], idx_d.at[b], semi[b])

    def wait_idx(b):
        pltpu.make_async_copy(edge_hbm.at[0, pl.ds(0, B)], idx_s.at[b],
                              semi[b]).wait()
        pltpu.make_async_copy(edge_hbm.at[1, pl.ds(0, B)], idx_d.at[b],
                              semi[b]).wait()

    def issue_gather(b):
        pltpu.async_copy(x_hbm.at[idx_s.at[b]], rows.at[b], semg[b])

    def wait_gather(b):
        pltpu.make_async_copy(x_hbm.at[idx_s.at[b]], rows.at[b],
                              semg[b]).wait()

    def section(b, kk, chunk):
        @pl.when(chunk < NCHUNK)
        def _():
            nb = 1 - b

            @pl.when(chunk + NW < NCHUNK)
            def _():
                wait_idx(nb)
                issue_gather(nb)
            for j in range(B // 16):
                d16 = idx_d[b, pl.ds(j * 16, 16)]
                plsc.addupdate_scatter(cnt, [d16], ones16)
            wait_gather(b)
            pltpu.sync_copy(rows.at[b], acc.at[idx_d.at[b]], add=True)

            @pl.when(chunk + 2 * NW < NCHUNK)
            def _():
                issue_idx(b, kk + 2)

    issue_idx(0, 0)
    wait_idx(0)
    issue_gather(0)

    @pl.when(wid + NW < NCHUNK)
    def _():
        issue_idx(1, 1)

    def outer(t, carry):
        kk0 = 2 * t
        section(0, kk0, wid + kk0 * NW)
        section(1, kk0 + 1, wid + (kk0 + 1) * NW)
        return carry
    lax.fori_loop(0, (KMAX + 1) // 2, outer, 0)

    cnt_out = pltpu.async_copy(cnt, cnt_hbm.at[wid], semi0)

    plsc.subcore_barrier()

    for j in range(ROWS_PAD // NS // ZCH):
        b = j % 2
        r0 = s * (ROWS_PAD // NS) + j * ZCH
        if j >= 2:
            r0p = s * (ROWS_PAD // NS) + (j - 2) * ZCH
            pltpu.make_async_copy(rows.at[b],
                                  out_hbm.at[c, pl.ds(r0p, ZCH), :],
                                  semg[b]).wait()
        pltpu.sync_copy(acc.at[pl.ds(r0, ZCH), :], rows.at[b])
        pltpu.async_copy(rows.at[b], out_hbm.at[c, pl.ds(r0, ZCH), :],
                         semg[b])
    for j in (3, 4):
        b = j % 2
        r0 = s * (ROWS_PAD // NS) + j * ZCH
        pltpu.make_async_copy(rows.at[b], out_hbm.at[c, pl.ds(r0, ZCH), :],
                              semg[b]).wait()
    cnt_out.wait()


def _finish_body(a_ref, c_ref, w_ref, b_ref, o_ref):
    a = a_ref[0] + a_ref[1]
    cnt = jnp.sum(c_ref[...], axis=0)[:, None]
    y = jnp.dot(a, w_ref[...], preferred_element_type=jnp.float32)
    y = y * lax.rsqrt(1.0 + cnt) + b_ref[...]
    o_ref[...] = jnp.maximum(y, 0.0)


RBLK = 5120

_finish = pl.pallas_call(
    _finish_body,
    grid=(ROWS_PAD // RBLK,),
    in_specs=[
        pl.BlockSpec((NC, RBLK, CH), lambda i: (0, i, 0)),
        pl.BlockSpec((NW, RBLK), lambda i: (0, i)),
        pl.BlockSpec((CH, CH), lambda i: (0, 0)),
        pl.BlockSpec((1, CH), lambda i: (0, 0)),
    ],
    out_specs=pl.BlockSpec((RBLK, CH), lambda i: (i, 0)),
    out_shape=jax.ShapeDtypeStruct((ROWS_PAD, CH), jnp.float32),
)


def kernel(x, edge_index, W, b):
    x = x.astype(jnp.float32)
    partials, counts = _sc_aggregate(x, edge_index.astype(jnp.int32))
    out = _finish(partials, counts,
                  W.astype(jnp.float32), b.astype(jnp.float32).reshape(1, CH))
    return out[:N_DST]
